# baseline (speedup 1.0000x reference)
import functools

import jax
import jax.numpy as jnp
from jax import lax
from jax.experimental import pallas as pl
from jax.experimental.pallas import tpu as pltpu
from jax.experimental.pallas import tpu_sc as plsc

N = 10000
E = 320000
D = 128

NC = 2
NS = 16
NW = NC * NS
EPW = E // NW
L = 16
NP = 10112
RPS = NP // NS
CW = 16

FCH = 128
EPWP = 10240
FNCH = EPWP // FCH
BLK = 8
NBLK = FNCH // BLK

CH = 100
NCH = EPW // CH

_SC_LINEAR = pltpu.CompilerParams(use_tc_tiling_on_sc=False,
                                  needs_layout_passes=False)
_SC_TILED = pltpu.CompilerParams(use_tc_tiling_on_sc=True,
                                 needs_layout_passes=False)


def _zero_rows(buf, nrows):
  width = buf.shape[1]
  zeros16 = jnp.zeros((L,), jnp.float32)

  def body(i, carry):
    r = i // (width // L)
    c = lax.rem(i, width // L)
    buf[r, pl.ds(c * L, L)] = zeros16
    return carry

  lax.fori_loop(0, nrows * width // L, body, 0)


def _zero_acc_slice(buf, acc, base):
  for j in range(RPS // 96):
    pltpu.sync_copy(buf.at[pl.ds(0, 96)], acc.at[pl.ds(base + 96 * j, 96)])
  rem = RPS % 96
  if rem:
    pltpu.sync_copy(buf.at[pl.ds(0, rem)],
                    acc.at[pl.ds(base + RPS - rem, rem)])


def _sc_feats(x, src, dst):
  mesh = plsc.VectorSubcoreMesh(core_axis_name="c", subcore_axis_name="s")

  @functools.partial(
      pl.kernel,
      out_type=jax.ShapeDtypeStruct((NC, NP, D), jnp.float32),
      mesh=mesh,
      scratch_types=[
          pltpu.VMEM((2 * BLK, FCH), jnp.int32),
          pltpu.VMEM((2 * BLK, FCH), jnp.int32),
          pltpu.VMEM((FCH, D), jnp.float32),
          pltpu.VMEM((FCH, D), jnp.float32),
          pltpu.VMEM_SHARED((NP, D), jnp.float32),
          [pltpu.SemaphoreType.DMA] * 4,
      ],
      compiler_params=_SC_TILED,
  )
  def k(x_hbm, src_hbm, dst_hbm, out_hbm,
        src_v, dst_v, buf0, buf1, acc_sh, sems):
    (gs0, gs1, is_s, is_d) = sems
    cid = lax.axis_index("c")
    sid = lax.axis_index("s")
    wid = sid * NC + cid

    _zero_rows(buf0, 96)
    _zero_acc_slice(buf0, acc_sh, sid * RPS)

    def iload(blk, rows):
      pltpu.async_copy(src_hbm.at[wid, pl.ds(BLK * blk, BLK)],
                       src_v.at[pl.ds(rows, BLK)], is_s)
      pltpu.async_copy(dst_hbm.at[wid, pl.ds(BLK * blk, BLK)],
                       dst_v.at[pl.ds(rows, BLK)], is_d)

    def iwait(blk, rows):
      pltpu.make_async_copy(src_hbm.at[wid, pl.ds(BLK * blk, BLK)],
                            src_v.at[pl.ds(rows, BLK)], is_s).wait()
      pltpu.make_async_copy(dst_hbm.at[wid, pl.ds(BLK * blk, BLK)],
                            dst_v.at[pl.ds(rows, BLK)], is_d).wait()

    def gather(row, buf, sem):
      pltpu.async_copy(x_hbm.at[src_v.at[row]], buf, sem)

    def gwait(row, buf, sem):
      pltpu.make_async_copy(x_hbm.at[src_v.at[row]], buf, sem).wait()

    def scatter(row, buf):
      pltpu.sync_copy(buf, acc_sh.at[dst_v.at[row]], add=True)

    def row(t, j):
      return 8 * lax.rem(t, 2) + j

    pltpu.sync_copy(src_hbm.at[wid, pl.ds(0, BLK)], src_v.at[pl.ds(0, BLK)])
    pltpu.sync_copy(dst_hbm.at[wid, pl.ds(0, BLK)], dst_v.at[pl.ds(0, BLK)])
    plsc.subcore_barrier()
    gather(0, buf0, gs0)
    iload(1, BLK)

    def body(t, carry):
      for j in range(0, BLK, 2):
        gather(row(t, j) + 1, buf1, gs1)
        gwait(row(t, j), buf0, gs0)
        scatter(row(t, j), buf0)
        if j == BLK - 2:
          iwait(t + 1, 8 - 8 * lax.rem(t, 2))
          gather(row(t + 1, 0), buf0, gs0)
        else:
          gather(row(t, j) + 2, buf0, gs0)
        gwait(row(t, j) + 1, buf1, gs1)
        scatter(row(t, j) + 1, buf1)
      iload(t + 2, 8 * lax.rem(t, 2))
      return carry

    lax.fori_loop(0, NBLK - 2, body, 0)

    for j in range(0, BLK, 2):
      gather(j + 1, buf1, gs1)
      gwait(j, buf0, gs0)
      scatter(j, buf0)
      if j == BLK - 2:
        iwait(NBLK - 1, BLK)
        gather(BLK, buf0, gs0)
      else:
        gather(j + 2, buf0, gs0)
      gwait(j + 1, buf1, gs1)
      scatter(j + 1, buf1)
    for j in range(0, BLK, 2):
      if j < BLK - 2:
        gather(BLK + j + 1, buf1, gs1)
        gwait(BLK + j, buf0, gs0)
        scatter(BLK + j, buf0)
        gather(BLK + j + 2, buf0, gs0)
        gwait(BLK + j + 1, buf1, gs1)
        scatter(BLK + j + 1, buf1)
      else:
        gather(BLK + j + 1, buf1, gs1)
        gwait(BLK + j, buf0, gs0)
        scatter(BLK + j, buf0)
        gwait(BLK + j + 1, buf1, gs1)
        scatter(BLK + j + 1, buf1)

    plsc.subcore_barrier()
    pltpu.sync_copy(acc_sh.at[pl.ds(sid * RPS, RPS)],
                    out_hbm.at[cid, pl.ds(sid * RPS, RPS)])

  return k(x, src, dst)


def _sc_counts(dst):
  mesh = plsc.VectorSubcoreMesh(core_axis_name="c", subcore_axis_name="s")

  @functools.partial(
      pl.kernel,
      out_type=jax.ShapeDtypeStruct((NC, NP, CW), jnp.float32),
      mesh=mesh,
      scratch_types=[
          pltpu.VMEM((NCH, CH), jnp.int32),
          pltpu.VMEM((CH, CW), jnp.float32),
          pltpu.VMEM((96, CW), jnp.float32),
          pltpu.VMEM_SHARED((NP, CW), jnp.float32),
          pltpu.SemaphoreType.DMA,
      ],
      compiler_params=_SC_LINEAR,
  )
  def k(dst_hbm, out_hbm, dst_v, ones_v, zbuf, acc_sh, sem):
    cid = lax.axis_index("c")
    sid = lax.axis_index("s")
    wid = sid * NC + cid
    ones16 = jnp.ones((L,), jnp.float32)

    def fill_ones(i, carry):
      ones_v[i, pl.ds(0, L)] = ones16
      return carry
    lax.fori_loop(0, CH, fill_ones, 0)
    _zero_rows(zbuf, 96)
    _zero_acc_slice(zbuf, acc_sh, sid * RPS)
    pltpu.sync_copy(dst_hbm.at[wid], dst_v)
    plsc.subcore_barrier()

    def fire(c, carry):
      pltpu.async_copy(ones_v, acc_sh.at[dst_v.at[c]], sem, add=True)
      return carry

    def drain(c, carry):
      pltpu.make_async_copy(ones_v, acc_sh.at[dst_v.at[c]], sem).wait()
      return carry

    lax.fori_loop(0, NCH, fire, 0)
    lax.fori_loop(0, NCH, drain, 0)

    plsc.subcore_barrier()
    pltpu.sync_copy(acc_sh.at[pl.ds(sid * RPS, RPS)],
                    out_hbm.at[cid, pl.ds(sid * RPS, RPS)])

  return k(dst)


def _tc_dense(acc, cacc, w_t, gamma, beta):
  def body(acc_ref, c_ref, w_ref, g_ref, b_ref, out_ref):
    s = acc_ref[0, :N] + acc_ref[1, :N]
    cnt = c_ref[0, :N, 0:1] + c_ref[1, :N, 0:1]
    mean = s / jnp.maximum(cnt, 1.0)
    h = jnp.dot(mean, w_ref[...], preferred_element_type=jnp.float32)
    h = jnp.where(h > 0, h, jnp.exp(jnp.minimum(h, 0.0)) - 1.0)
    mu = jnp.mean(h, axis=0, keepdims=True)
    xc = h - mu
    var = jnp.mean(xc * xc, axis=0, keepdims=True)
    out_ref[...] = g_ref[...] * (xc * lax.rsqrt(var + 1e-5)) + b_ref[...]

  return pl.pallas_call(
      body,
      out_shape=jax.ShapeDtypeStruct((N, D), jnp.float32),
  )(acc, cacc, w_t, gamma, beta)


def kernel(x, edge_index, W, gamma, beta):
  src2 = edge_index[0].reshape(NW, EPW)
  dst2 = edge_index[1].reshape(NW, EPW)
  srcp = jnp.pad(src2, ((0, 0), (0, EPWP - EPW))).reshape(NW, FNCH, FCH)
  dstp = jnp.pad(dst2, ((0, 0), (0, EPWP - EPW)),
                 constant_values=N).reshape(NW, FNCH, FCH)
  acc = _sc_feats(x, srcp, dstp)
  cacc = _sc_counts(dst2.reshape(NW, NCH, CH))
  return _tc_dense(acc, cacc, W.T, gamma.reshape(1, D), beta.reshape(1, D))


# speedup vs baseline: 2.6537x; 2.6537x over previous
import functools

import jax
import jax.numpy as jnp
from jax import lax
from jax.experimental import pallas as pl
from jax.experimental.pallas import tpu as pltpu
from jax.experimental.pallas import tpu_sc as plsc

N = 10000
E = 320000
D = 128

NC = 2
NS = 16
NW = NC * NS
EPW = E // NW
CH = 100
NCH = EPW // CH
NP = 10112
RPS = NP // NS
CW = 16
L = 16

_SC_PARAMS = pltpu.CompilerParams(use_tc_tiling_on_sc=False,
                                  needs_layout_passes=False)


def _zero_rows(buf, nrows):
  width = buf.shape[1]
  zeros16 = jnp.zeros((L,), jnp.float32)

  def body(i, carry):
    r = i // (width // L)
    c = lax.rem(i, width // L)
    buf[r, pl.ds(c * L, L)] = zeros16
    return carry

  lax.fori_loop(0, nrows * width // L, body, 0)


def _zero_acc_slice(buf, acc, base):
  for j in range(RPS // 96):
    pltpu.sync_copy(buf.at[pl.ds(0, 96)], acc.at[pl.ds(base + 96 * j, 96)])
  rem = RPS % 96
  if rem:
    pltpu.sync_copy(buf.at[pl.ds(0, rem)],
                    acc.at[pl.ds(base + RPS - rem, rem)])


def _sc_feats(x, src, dst):
  mesh = plsc.VectorSubcoreMesh(core_axis_name="c", subcore_axis_name="s")

  @functools.partial(
      pl.kernel,
      out_type=jax.ShapeDtypeStruct((NC, NP, D), jnp.float32),
      mesh=mesh,
      scratch_types=[
          pltpu.VMEM((NCH, CH), jnp.int32),
          pltpu.VMEM((NCH, CH), jnp.int32),
          pltpu.VMEM((CH, D), jnp.float32),
          pltpu.VMEM((CH, D), jnp.float32),
          pltpu.VMEM_SHARED((NP, D), jnp.float32),
          pltpu.SemaphoreType.DMA,
          pltpu.SemaphoreType.DMA,
      ],
      compiler_params=_SC_PARAMS,
  )
  def k(x_hbm, src_hbm, dst_hbm, out_hbm,
        src_v, dst_v, buf0, buf1, acc_sh, sem0, sem1):
    cid = lax.axis_index("c")
    sid = lax.axis_index("s")
    wid = sid * NC + cid

    _zero_rows(buf0, 96)
    _zero_acc_slice(buf0, acc_sh, sid * RPS)
    pltpu.sync_copy(src_hbm.at[wid], src_v)
    pltpu.sync_copy(dst_hbm.at[wid], dst_v)
    plsc.subcore_barrier()

    def gather(c, buf, sem):
      pltpu.async_copy(x_hbm.at[src_v.at[c]], buf, sem)

    def gwait(c, buf, sem):
      pltpu.make_async_copy(x_hbm.at[src_v.at[c]], buf, sem).wait()

    def scatter(c, buf):
      pltpu.sync_copy(buf, acc_sh.at[dst_v.at[c]], add=True)

    gather(0, buf0, sem0)

    def body(t, carry):
      c = 2 * t
      gather(c + 1, buf1, sem1)
      gwait(c, buf0, sem0)
      scatter(c, buf0)
      gather(c + 2, buf0, sem0)
      gwait(c + 1, buf1, sem1)
      scatter(c + 1, buf1)
      return carry

    lax.fori_loop(0, NCH // 2 - 1, body, 0)
    c = NCH - 2
    gather(c + 1, buf1, sem1)
    gwait(c, buf0, sem0)
    scatter(c, buf0)
    gwait(c + 1, buf1, sem1)
    scatter(c + 1, buf1)

    plsc.subcore_barrier()
    pltpu.sync_copy(acc_sh.at[pl.ds(sid * RPS, RPS)],
                    out_hbm.at[cid, pl.ds(sid * RPS, RPS)])

  return k(x, src, dst)


def _sc_counts(dst):
  mesh = plsc.VectorSubcoreMesh(core_axis_name="c", subcore_axis_name="s")

  @functools.partial(
      pl.kernel,
      out_type=jax.ShapeDtypeStruct((NC, NP, CW), jnp.float32),
      mesh=mesh,
      scratch_types=[
          pltpu.VMEM((NCH, CH), jnp.int32),
          pltpu.VMEM((CH, CW), jnp.float32),
          pltpu.VMEM((96, CW), jnp.float32),
          pltpu.VMEM_SHARED((NP, CW), jnp.float32),
          pltpu.SemaphoreType.DMA,
      ],
      compiler_params=_SC_PARAMS,
  )
  def k(dst_hbm, out_hbm, dst_v, ones_v, zbuf, acc_sh, sem):
    cid = lax.axis_index("c")
    sid = lax.axis_index("s")
    wid = sid * NC + cid
    ones16 = jnp.ones((L,), jnp.float32)

    def fill_ones(i, carry):
      ones_v[i, pl.ds(0, L)] = ones16
      return carry
    lax.fori_loop(0, CH, fill_ones, 0)
    _zero_rows(zbuf, 96)
    _zero_acc_slice(zbuf, acc_sh, sid * RPS)
    pltpu.sync_copy(dst_hbm.at[wid], dst_v)
    plsc.subcore_barrier()

    def fire(c, carry):
      pltpu.async_copy(ones_v, acc_sh.at[dst_v.at[c]], sem, add=True)
      return carry

    def drain(c, carry):
      pltpu.make_async_copy(ones_v, acc_sh.at[dst_v.at[c]], sem).wait()
      return carry

    lax.fori_loop(0, NCH, fire, 0)
    lax.fori_loop(0, NCH, drain, 0)

    plsc.subcore_barrier()
    pltpu.sync_copy(acc_sh.at[pl.ds(sid * RPS, RPS)],
                    out_hbm.at[cid, pl.ds(sid * RPS, RPS)])

  return k(dst)


def _tc_dense(acc, cacc, w_t, gamma, beta):
  def body(acc_ref, c_ref, w_ref, g_ref, b_ref, out_ref):
    s = acc_ref[0, :N] + acc_ref[1, :N]
    cnt = c_ref[0, :N, 0:1] + c_ref[1, :N, 0:1]
    mean = s / jnp.maximum(cnt, 1.0)
    h = jnp.dot(mean, w_ref[...], preferred_element_type=jnp.float32)
    h = jnp.where(h > 0, h, jnp.exp(jnp.minimum(h, 0.0)) - 1.0)
    mu = jnp.mean(h, axis=0, keepdims=True)
    xc = h - mu
    var = jnp.mean(xc * xc, axis=0, keepdims=True)
    out_ref[...] = g_ref[...] * (xc * lax.rsqrt(var + 1e-5)) + b_ref[...]

  return pl.pallas_call(
      body,
      out_shape=jax.ShapeDtypeStruct((N, D), jnp.float32),
  )(acc, cacc, w_t, gamma, beta)


def kernel(x, edge_index, W, gamma, beta):
  src = edge_index[0].reshape(NW, NCH, CH)
  dst = edge_index[1].reshape(NW, NCH, CH)
  acc = _sc_feats(x, src, dst)
  cacc = _sc_counts(dst)
  return _tc_dense(acc, cacc, W.T, gamma.reshape(1, D), beta.reshape(1, D))
